# Initial kernel scaffold; baseline (speedup 1.0000x reference)
import jax
import jax.numpy as jnp
from jax.experimental import pallas as pl


def kernel(position_sequence, params, particle_types, senders, receivers, n_particles_per_example):
    raise NotImplementedError("write your pallas kernel here")


# speedup vs baseline: 1.7342x; 1.7342x over previous
import functools

import jax
import jax.numpy as jnp
from jax import lax
from jax.experimental import pallas as pl
from jax.experimental.pallas import tpu as pltpu
from jax.experimental.pallas import tpu_sc as plsc

_N = 10000
_E = 160000
_DIMS = 3
_L = 64
_RADIUS = 0.015
_NTYPES = 9
_MP_STEPS = 5

_NC = 2
_NS = 16
_NW = _NC * _NS
_IDXW = 128
_CPW = 40
_EPAD = _NW * _CPW * _IDXW
_KB = 8
_NBLK = _CPW // _KB
_NACC = _N + 8
_RPS = _N // _NS

_MESH = plsc.VectorSubcoreMesh(core_axis_name="c", subcore_axis_name="s")
_SC_PARAMS = pltpu.CompilerParams(use_tc_tiling_on_sc=False)


def _gather2_body(ts_ref, tr_ref, si_ref, ri_ref, gs_ref, gr_ref,
                  idx_v, rows_v, sem):
    wid = lax.axis_index("s") * _NC + lax.axis_index("c")
    for idx_hbm, tbl, out in ((si_ref, ts_ref, gs_ref), (ri_ref, tr_ref, gr_ref)):
        for k in range(_NBLK):
            row0 = wid * _CPW + k * _KB
            pltpu.sync_copy(idx_hbm.at[pl.ds(row0, _KB)], idx_v)
            cps = [pltpu.async_copy(tbl.at[idx_v.at[j]],
                                    rows_v.at[pl.ds(j * _IDXW, _IDXW)], sem)
                   for j in range(_KB)]
            for c in cps:
                c.wait()
            pltpu.sync_copy(rows_v, out.at[pl.ds(row0 * _IDXW, _KB * _IDXW)])


def _gather2(d, tbl_s, tbl_r, sidx, ridx):
    fn = pl.kernel(
        _gather2_body,
        out_type=[jax.ShapeDtypeStruct((_EPAD, d), jnp.float32)] * 2,
        mesh=_MESH,
        compiler_params=_SC_PARAMS,
        scratch_types=[
            pltpu.VMEM((_KB, _IDXW), jnp.int32),
            pltpu.VMEM((_KB * _IDXW, d), jnp.float32),
            pltpu.SemaphoreType.DMA,
        ],
    )
    return fn(tbl_s, tbl_r, sidx, ridx)


def _scatter_body(en_ref, ri_ref, z_ref, out_ref, idx_v, rows_v, sem, acc_sh):
    cid = lax.axis_index("c")
    sid = lax.axis_index("s")
    wid = sid * _NC + cid
    pltpu.sync_copy(z_ref.at[pl.ds(sid * _RPS, _RPS)],
                    acc_sh.at[pl.ds(sid * _RPS, _RPS)])
    plsc.subcore_barrier()
    for k in range(_NBLK):
        row0 = wid * _CPW + k * _KB
        pltpu.sync_copy(ri_ref.at[pl.ds(row0, _KB)], idx_v)
        pltpu.sync_copy(en_ref.at[pl.ds(row0 * _IDXW, _KB * _IDXW)], rows_v)
        for j in range(_KB):
            pltpu.sync_copy(rows_v.at[pl.ds(j * _IDXW, _IDXW)],
                            acc_sh.at[idx_v.at[j]], add=True)
    plsc.subcore_barrier()
    pltpu.sync_copy(acc_sh.at[pl.ds(sid * _RPS, _RPS)],
                    out_ref.at[cid, pl.ds(sid * _RPS, _RPS)])


def _scatter(en, ridx, zeros):
    fn = pl.kernel(
        _scatter_body,
        out_type=jax.ShapeDtypeStruct((_NC, _N, _L), jnp.float32),
        mesh=_MESH,
        compiler_params=_SC_PARAMS,
        scratch_types=[
            pltpu.VMEM((_KB, _IDXW), jnp.int32),
            pltpu.VMEM((_KB * _IDXW, _L), jnp.float32),
            pltpu.SemaphoreType.DMA,
            pltpu.VMEM_SHARED((_NACC, _L), jnp.float32),
        ],
    )
    return fn(en, ridx, zeros)


_BE = 2048
_BN = 2000


def _ln(x):
    m = jnp.mean(x, axis=-1, keepdims=True)
    xc = x - m
    var = jnp.mean(xc * xc, axis=-1, keepdims=True)
    return xc * lax.rsqrt(var + 1e-6)


def _dot(a, b):
    return jnp.dot(a, b, preferred_element_type=jnp.float32)


def _edge_body(e_ref, gs_ref, gr_ref, w1_ref, b1_ref, w2_ref, b2_ref,
               eo_ref, en_ref):
    pre = _dot(e_ref[...], w1_ref[...]) + gs_ref[...] + gr_ref[...] + b1_ref[...]
    h = jnp.maximum(pre, 0.0)
    en = _ln(_dot(h, w2_ref[...]) + b2_ref[...])
    en_ref[...] = en
    eo_ref[...] = e_ref[...] + en


def _edge_step(e, gs, gr, w1, b1, w2, b2):
    blk = lambda r, c: pl.BlockSpec((r, c), lambda i: (i, 0))
    cst = lambda r, c: pl.BlockSpec((r, c), lambda i: (0, 0))
    return pl.pallas_call(
        _edge_body,
        grid=(_EPAD // _BE,),
        in_specs=[blk(_BE, _L), blk(_BE, _L), blk(_BE, _L),
                  cst(_L, _L), cst(1, _L), cst(_L, _L), cst(1, _L)],
        out_specs=[blk(_BE, _L), blk(_BE, _L)],
        out_shape=[jax.ShapeDtypeStruct((_EPAD, _L), jnp.float32)] * 2,
    )(e, gs, gr, w1, b1, w2, b2)


def _enc_edge_body(sp_ref, rp_ref, w1_ref, wn_ref, b1_ref, w2_ref, b2_ref,
                   e_ref):
    rel = (sp_ref[...] - rp_ref[...]) * (1.0 / _RADIUS)
    nrm = jnp.sqrt(jnp.sum(rel * rel, axis=-1, keepdims=True))
    pre = _dot(rel, w1_ref[...]) + nrm * wn_ref[...] + b1_ref[...]
    h = jnp.maximum(pre, 0.0)
    e_ref[...] = _ln(_dot(h, w2_ref[...]) + b2_ref[...])


def _enc_edge(spos, rpos, w1p, wn, b1, w2, b2):
    blk = lambda r, c: pl.BlockSpec((r, c), lambda i: (i, 0))
    cst = lambda r, c: pl.BlockSpec((r, c), lambda i: (0, 0))
    return pl.pallas_call(
        _enc_edge_body,
        grid=(_EPAD // _BE,),
        in_specs=[blk(_BE, 16), blk(_BE, 16),
                  cst(16, _L), cst(1, _L), cst(1, _L), cst(_L, _L), cst(1, _L)],
        out_specs=blk(_BE, _L),
        out_shape=jax.ShapeDtypeStruct((_EPAD, _L), jnp.float32),
    )(spos, rpos, w1p, wn, b1, w2, b2)


def _enc_node_body(x_ref, w1_ref, b1_ref, w2_ref, b2_ref, ws_ref, wr_ref,
                   v_ref, ps_ref, pr_ref):
    h = jnp.maximum(_dot(x_ref[...], w1_ref[...]) + b1_ref[...], 0.0)
    v = _ln(_dot(h, w2_ref[...]) + b2_ref[...])
    v_ref[...] = v
    ps_ref[...] = _dot(v, ws_ref[...])
    pr_ref[...] = _dot(v, wr_ref[...])


def _enc_node(x, w1, b1, w2, b2, ws, wr):
    blk = lambda r, c: pl.BlockSpec((r, c), lambda i: (i, 0))
    cst = lambda r, c: pl.BlockSpec((r, c), lambda i: (0, 0))
    return pl.pallas_call(
        _enc_node_body,
        grid=(_N // _BN,),
        in_specs=[blk(_BN, 32), cst(32, _L), cst(1, _L), cst(_L, _L),
                  cst(1, _L), cst(_L, _L), cst(_L, _L)],
        out_specs=[blk(_BN, _L)] * 3,
        out_shape=[jax.ShapeDtypeStruct((_N, _L), jnp.float32)] * 3,
    )(x, w1, b1, w2, b2, ws, wr)


def _node_mid_body(v_ref, p0_ref, p1_ref, uv_ref, ua_ref, c1_ref, u2_ref,
                   c2_ref, ws_ref, wr_ref, vo_ref, ps_ref, pr_ref):
    v = v_ref[...]
    agg = p0_ref[...] + p1_ref[...]
    g = jnp.maximum(_dot(v, uv_ref[...]) + _dot(agg, ua_ref[...]) + c1_ref[...],
                    0.0)
    vo = v + _ln(_dot(g, u2_ref[...]) + c2_ref[...])
    vo_ref[...] = vo
    ps_ref[...] = _dot(vo, ws_ref[...])
    pr_ref[...] = _dot(vo, wr_ref[...])


def _node_mid(v, p0, p1, uv, ua, c1, u2, c2, ws, wr):
    blk = lambda r, c: pl.BlockSpec((r, c), lambda i: (i, 0))
    cst = lambda r, c: pl.BlockSpec((r, c), lambda i: (0, 0))
    return pl.pallas_call(
        _node_mid_body,
        grid=(_N // _BN,),
        in_specs=[blk(_BN, _L), blk(_BN, _L), blk(_BN, _L),
                  cst(_L, _L), cst(_L, _L), cst(1, _L), cst(_L, _L),
                  cst(1, _L), cst(_L, _L), cst(_L, _L)],
        out_specs=[blk(_BN, _L)] * 3,
        out_shape=[jax.ShapeDtypeStruct((_N, _L), jnp.float32)] * 3,
    )(v, p0, p1, uv, ua, c1, u2, c2, ws, wr)


def _node_fin_body(v_ref, p0_ref, p1_ref, uv_ref, ua_ref, c1_ref, u2_ref,
                   c2_ref, d1_ref, e1_ref, d2_ref, e2_ref, base_ref, out_ref):
    v = v_ref[...]
    agg = p0_ref[...] + p1_ref[...]
    g = jnp.maximum(_dot(v, uv_ref[...]) + _dot(agg, ua_ref[...]) + c1_ref[...],
                    0.0)
    vo = v + _ln(_dot(g, u2_ref[...]) + c2_ref[...])
    hd = jnp.maximum(_dot(vo, d1_ref[...]) + e1_ref[...], 0.0)
    acc = _dot(hd, d2_ref[...]) + e2_ref[...]
    out_ref[...] = base_ref[...] + acc


def _node_fin(v, p0, p1, uv, ua, c1, u2, c2, d1, e1, d2, e2, base):
    blk = lambda r, c: pl.BlockSpec((r, c), lambda i: (i, 0))
    cst = lambda r, c: pl.BlockSpec((r, c), lambda i: (0, 0))
    return pl.pallas_call(
        _node_fin_body,
        grid=(_N // _BN,),
        in_specs=[blk(_BN, _L), blk(_BN, _L), blk(_BN, _L),
                  cst(_L, _L), cst(_L, _L), cst(1, _L), cst(_L, _L),
                  cst(1, _L), cst(_L, _L), cst(1, _L), cst(_L, _L),
                  cst(1, _L), blk(_BN, _L)],
        out_specs=blk(_BN, _L),
        out_shape=jax.ShapeDtypeStruct((_N, _L), jnp.float32),
    )(v, p0, p1, uv, ua, c1, u2, c2, d1, e1, d2, e2, base)


def kernel(position_sequence, params, particle_types, senders, receivers,
           n_particles_per_example):
    f32 = jnp.float32
    pos = position_sequence
    most = pos[:, -1]

    nvel = (pos[:, 1:] - pos[:, :-1]).reshape(_N, -1)
    dist = jnp.concatenate([most - 0.1, 0.9 - most], axis=1)
    dist = jnp.clip(dist * (1.0 / _RADIUS), -1.0, 1.0)
    onehot = (particle_types[:, None]
              == jnp.arange(_NTYPES, dtype=particle_types.dtype)[None, :])
    x = jnp.concatenate([nvel, dist, onehot.astype(f32),
                         jnp.zeros((_N, 2), f32)], axis=1)

    a1w, a1b = params['enc_node'][0]
    a2w, a2b = params['enc_node'][1]
    wn1 = jnp.concatenate(
        [a1w[:21], params['type_emb'] @ a1w[21:37], jnp.zeros((2, _L), f32)],
        axis=0)
    b1w, b1b = params['enc_edge'][0]
    b2w, b2b = params['enc_edge'][1]
    be1 = jnp.concatenate([b1w[:3], jnp.zeros((13, _L), f32)], axis=0)
    ben = b1w[3:4]
    dw1, db1 = params['dec'][0]
    dw2, db2 = params['dec'][1]
    dw2p = jnp.concatenate([dw2, jnp.zeros((_L, _L - _DIMS), f32)], axis=1)
    db2p = jnp.concatenate([db2, jnp.zeros((_L - _DIMS,), f32)])[None, :]

    steps = []
    for sp in params['proc']:
        w1, bb1 = sp['edge'][0]
        w2, bb2 = sp['edge'][1]
        u1, cc1 = sp['node'][0]
        u2, cc2 = sp['node'][1]
        steps.append(dict(
            w1e=w1[:_L], w1s=w1[_L:2 * _L], w1r=w1[2 * _L:], b1=bb1[None, :],
            w2=w2, b2=bb2[None, :], uv=u1[:_L], ua=u1[_L:], c1=cc1[None, :],
            u2=u2, c2=cc2[None, :]))

    npad = _EPAD - _E
    i32 = jnp.int32
    s_i = senders.astype(i32)
    r_i = receivers.astype(i32)
    sidx = jnp.concatenate([s_i, jnp.zeros((npad,), i32)]).reshape(-1, _IDXW)
    ridx_g = jnp.concatenate([r_i, jnp.zeros((npad,), i32)]).reshape(-1, _IDXW)
    ridx_s = jnp.concatenate([r_i, jnp.full((npad,), _N, i32)]).reshape(-1, _IDXW)
    zeros_nl = jnp.zeros((_N, _L), f32)

    tpos = jnp.concatenate([most, jnp.zeros((_N, 13), f32)], axis=1)

    spos, rpos = _gather2(16, tpos, tpos, sidx, ridx_g)
    e = _enc_edge(spos, rpos, be1, ben, b1b[None, :], b2w, b2b[None, :])
    v, ps, pr = _enc_node(x, wn1, a1b[None, :], a2w, a2b[None, :],
                          steps[0]['w1s'], steps[0]['w1r'])

    base = jnp.concatenate(
        [2.0 * most - pos[:, -2], jnp.zeros((_N, _L - _DIMS), f32)], axis=1)

    for t in range(_MP_STEPS):
        st = steps[t]
        gs, gr = _gather2(_L, ps, pr, sidx, ridx_g)
        e, en = _edge_step(e, gs, gr, st['w1e'], st['b1'], st['w2'], st['b2'])
        parts = _scatter(en, ridx_s, zeros_nl)
        if t < _MP_STEPS - 1:
            nx = steps[t + 1]
            v, ps, pr = _node_mid(v, parts[0], parts[1], st['uv'], st['ua'],
                                  st['c1'], st['u2'], st['c2'],
                                  nx['w1s'], nx['w1r'])
        else:
            out = _node_fin(v, parts[0], parts[1], st['uv'], st['ua'],
                            st['c1'], st['u2'], st['c2'], dw1, db1[None, :],
                            dw2p, db2p, base)

    return out[:, :_DIMS]


# speedup vs baseline: 1.8194x; 1.0492x over previous
import functools

import jax
import jax.numpy as jnp
from jax import lax
from jax.experimental import pallas as pl
from jax.experimental.pallas import tpu as pltpu
from jax.experimental.pallas import tpu_sc as plsc

_N = 10000
_E = 160000
_DIMS = 3
_L = 64
_RADIUS = 0.015
_NTYPES = 9
_MP_STEPS = 5

_NC = 2
_NS = 16
_NW = _NC * _NS
_IDXW = 128
_CPW = 40
_EPAD = _NW * _CPW * _IDXW
_KB = 8
_NBLK = _CPW // _KB
_NACC = _N + 8
_RPS = _N // _NS

_MESH = plsc.VectorSubcoreMesh(core_axis_name="c", subcore_axis_name="s")
_SC_PARAMS = pltpu.CompilerParams(use_tc_tiling_on_sc=False)


_GCH = 4
_GRING = 3
_SRING = 2


def _gather2_body(ts_ref, tr_ref, si_ref, ri_ref, gs_ref, gr_ref,
                  idx_s, idx_r, bufs, g0, g1, g2, o0, o1, o2):
    gsem = (g0, g1, g2)
    osem = (o0, o1, o2)
    wid = lax.axis_index("s") * _NC + lax.axis_index("c")
    row0 = wid * _CPW
    pltpu.sync_copy(si_ref.at[pl.ds(row0, _CPW)], idx_s)
    pltpu.sync_copy(ri_ref.at[pl.ds(row0, _CPW)], idx_r)
    nch = _CPW // _GCH
    chunks = ([(idx_s, ts_ref, gs_ref, c) for c in range(nch)]
              + [(idx_r, tr_ref, gr_ref, c) for c in range(nch)])
    gds = [None] * _GRING
    ods = [None] * _GRING
    outinfo = [None] * _GRING
    for ci, (idx_v, tbl, out, c) in enumerate(chunks):
        b = ci % _GRING
        if ods[b] is not None:
            ods[b].wait()
        gds_b = [pltpu.async_copy(tbl.at[idx_v.at[c * _GCH + j]],
                                  bufs.at[b, pl.ds(j * _IDXW, _IDXW)],
                                  gsem[b])
                 for j in range(_GCH)]
        if ci >= 1:
            pb = (ci - 1) % _GRING
            for dsc in gds[pb]:
                dsc.wait()
            pout, prow = outinfo[pb]
            ods[pb] = pltpu.async_copy(bufs.at[pb],
                                       pout.at[pl.ds(prow, _GCH * _IDXW)],
                                       osem[pb])
        gds[b] = gds_b
        outinfo[b] = (out, (row0 + c * _GCH) * _IDXW)
    lb = (len(chunks) - 1) % _GRING
    for dsc in gds[lb]:
        dsc.wait()
    pout, prow = outinfo[lb]
    ods[lb] = pltpu.async_copy(bufs.at[lb], pout.at[pl.ds(prow, _GCH * _IDXW)],
                               osem[lb])
    for b in range(_GRING):
        if ods[b] is not None:
            ods[b].wait()


def _gather2(d, tbl_s, tbl_r, sidx, ridx):
    fn = pl.kernel(
        _gather2_body,
        out_type=[jax.ShapeDtypeStruct((_EPAD, d), jnp.float32)] * 2,
        mesh=_MESH,
        compiler_params=_SC_PARAMS,
        scratch_types=[
            pltpu.VMEM((_CPW, _IDXW), jnp.int32),
            pltpu.VMEM((_CPW, _IDXW), jnp.int32),
            pltpu.VMEM((_GRING, _GCH * _IDXW, d), jnp.float32),
        ] + [pltpu.SemaphoreType.DMA] * (2 * _GRING),
    )
    return fn(tbl_s, tbl_r, sidx, ridx)


_ZR = 64


def _scatter_body(en_ref, ri_ref, out_ref, idx_v, bufs, zbuf, l0, l1,
                  s0, s1, acc_sh):
    lsem = (l0, l1)
    ssem = (s0, s1)
    cid = lax.axis_index("c")
    sid = lax.axis_index("s")
    wid = sid * _NC + cid
    pltpu.sync_copy(ri_ref.at[pl.ds(wid * _CPW, _CPW)], idx_v)

    def _zb(i, _):
        zbuf[i // 4, pl.ds((i % 4) * 16, 16)] = jnp.zeros((16,), jnp.float32)
        return _

    lax.fori_loop(0, _ZR * 4, _zb, 0)
    nfull = _RPS // _ZR
    for zi in range(nfull):
        pltpu.sync_copy(zbuf, acc_sh.at[pl.ds(sid * _RPS + zi * _ZR, _ZR)])
    rem = _RPS - nfull * _ZR
    if rem:
        pltpu.sync_copy(zbuf.at[pl.ds(0, rem)],
                        acc_sh.at[pl.ds(sid * _RPS + nfull * _ZR, rem)])

    @pl.when(sid == 0)
    def _():
        pltpu.sync_copy(zbuf.at[pl.ds(0, _NACC - _N)],
                        acc_sh.at[pl.ds(_N, _NACC - _N)])

    plsc.subcore_barrier()
    nch = _CPW // _GCH
    lds = [None] * _SRING
    sds = [None] * _SRING
    for c in range(nch):
        b = c % _SRING
        if sds[b] is not None:
            for dsc in sds[b]:
                dsc.wait()
        row0 = wid * _CPW + c * _GCH
        lds[b] = pltpu.async_copy(
            en_ref.at[pl.ds(row0 * _IDXW, _GCH * _IDXW)], bufs.at[b],
            lsem[b])
        if c >= 1:
            pb = (c - 1) % _SRING
            lds[pb].wait()
            sds[pb] = [pltpu.async_copy(bufs.at[pb, pl.ds(j * _IDXW, _IDXW)],
                                        acc_sh.at[idx_v.at[(c - 1) * _GCH + j]],
                                        ssem[pb], add=True)
                       for j in range(_GCH)]
    lb = (nch - 1) % _SRING
    lds[lb].wait()
    sds[lb] = [pltpu.async_copy(bufs.at[lb, pl.ds(j * _IDXW, _IDXW)],
                                acc_sh.at[idx_v.at[(nch - 1) * _GCH + j]],
                                ssem[lb], add=True)
               for j in range(_GCH)]
    for b in range(_SRING):
        if sds[b] is not None:
            for dsc in sds[b]:
                dsc.wait()
    plsc.subcore_barrier()
    pltpu.sync_copy(acc_sh.at[pl.ds(sid * _RPS, _RPS)],
                    out_ref.at[cid, pl.ds(sid * _RPS, _RPS)])


def _scatter(en, ridx):
    fn = pl.kernel(
        _scatter_body,
        out_type=jax.ShapeDtypeStruct((_NC, _N, _L), jnp.float32),
        mesh=_MESH,
        compiler_params=_SC_PARAMS,
        scratch_types=[
            pltpu.VMEM((_CPW, _IDXW), jnp.int32),
            pltpu.VMEM((_SRING, _GCH * _IDXW, _L), jnp.float32),
            pltpu.VMEM((_ZR, _L), jnp.float32),
        ] + [pltpu.SemaphoreType.DMA] * (2 * _SRING) + [
            pltpu.VMEM_SHARED((_NACC, _L), jnp.float32),
        ],
    )
    return fn(en, ridx)


_BE = 2048
_BN = 2000


def _ln(x):
    m = jnp.mean(x, axis=-1, keepdims=True)
    xc = x - m
    var = jnp.mean(xc * xc, axis=-1, keepdims=True)
    return xc * lax.rsqrt(var + 1e-6)


def _dot(a, b):
    return jnp.dot(a, b, preferred_element_type=jnp.float32)


def _edge_body(e_ref, gs_ref, gr_ref, w1_ref, b1_ref, w2_ref, b2_ref,
               eo_ref, en_ref):
    pre = _dot(e_ref[...], w1_ref[...]) + gs_ref[...] + gr_ref[...] + b1_ref[...]
    h = jnp.maximum(pre, 0.0)
    en = _ln(_dot(h, w2_ref[...]) + b2_ref[...])
    en_ref[...] = en
    eo_ref[...] = e_ref[...] + en


def _edge_step(e, gs, gr, w1, b1, w2, b2):
    blk = lambda r, c: pl.BlockSpec((r, c), lambda i: (i, 0))
    cst = lambda r, c: pl.BlockSpec((r, c), lambda i: (0, 0))
    return pl.pallas_call(
        _edge_body,
        grid=(_EPAD // _BE,),
        in_specs=[blk(_BE, _L), blk(_BE, _L), blk(_BE, _L),
                  cst(_L, _L), cst(1, _L), cst(_L, _L), cst(1, _L)],
        out_specs=[blk(_BE, _L), blk(_BE, _L)],
        out_shape=[jax.ShapeDtypeStruct((_EPAD, _L), jnp.float32)] * 2,
    )(e, gs, gr, w1, b1, w2, b2)


def _enc_edge_body(sp_ref, rp_ref, w1_ref, wn_ref, b1_ref, w2_ref, b2_ref,
                   e_ref):
    rel = (sp_ref[...] - rp_ref[...]) * (1.0 / _RADIUS)
    nrm = jnp.sqrt(jnp.sum(rel * rel, axis=-1, keepdims=True))
    pre = _dot(rel, w1_ref[...]) + nrm * wn_ref[...] + b1_ref[...]
    h = jnp.maximum(pre, 0.0)
    e_ref[...] = _ln(_dot(h, w2_ref[...]) + b2_ref[...])


def _enc_edge(spos, rpos, w1p, wn, b1, w2, b2):
    blk = lambda r, c: pl.BlockSpec((r, c), lambda i: (i, 0))
    cst = lambda r, c: pl.BlockSpec((r, c), lambda i: (0, 0))
    return pl.pallas_call(
        _enc_edge_body,
        grid=(_EPAD // _BE,),
        in_specs=[blk(_BE, 16), blk(_BE, 16),
                  cst(16, _L), cst(1, _L), cst(1, _L), cst(_L, _L), cst(1, _L)],
        out_specs=blk(_BE, _L),
        out_shape=jax.ShapeDtypeStruct((_EPAD, _L), jnp.float32),
    )(spos, rpos, w1p, wn, b1, w2, b2)


def _enc_node_body(x_ref, w1_ref, b1_ref, w2_ref, b2_ref, ws_ref, wr_ref,
                   v_ref, ps_ref, pr_ref):
    h = jnp.maximum(_dot(x_ref[...], w1_ref[...]) + b1_ref[...], 0.0)
    v = _ln(_dot(h, w2_ref[...]) + b2_ref[...])
    v_ref[...] = v
    ps_ref[...] = _dot(v, ws_ref[...])
    pr_ref[...] = _dot(v, wr_ref[...])


def _enc_node(x, w1, b1, w2, b2, ws, wr):
    blk = lambda r, c: pl.BlockSpec((r, c), lambda i: (i, 0))
    cst = lambda r, c: pl.BlockSpec((r, c), lambda i: (0, 0))
    return pl.pallas_call(
        _enc_node_body,
        grid=(_N // _BN,),
        in_specs=[blk(_BN, 32), cst(32, _L), cst(1, _L), cst(_L, _L),
                  cst(1, _L), cst(_L, _L), cst(_L, _L)],
        out_specs=[blk(_BN, _L)] * 3,
        out_shape=[jax.ShapeDtypeStruct((_N, _L), jnp.float32)] * 3,
    )(x, w1, b1, w2, b2, ws, wr)


def _node_mid_body(v_ref, p0_ref, p1_ref, uv_ref, ua_ref, c1_ref, u2_ref,
                   c2_ref, ws_ref, wr_ref, vo_ref, ps_ref, pr_ref):
    v = v_ref[...]
    agg = p0_ref[...] + p1_ref[...]
    g = jnp.maximum(_dot(v, uv_ref[...]) + _dot(agg, ua_ref[...]) + c1_ref[...],
                    0.0)
    vo = v + _ln(_dot(g, u2_ref[...]) + c2_ref[...])
    vo_ref[...] = vo
    ps_ref[...] = _dot(vo, ws_ref[...])
    pr_ref[...] = _dot(vo, wr_ref[...])


def _node_mid(v, p0, p1, uv, ua, c1, u2, c2, ws, wr):
    blk = lambda r, c: pl.BlockSpec((r, c), lambda i: (i, 0))
    cst = lambda r, c: pl.BlockSpec((r, c), lambda i: (0, 0))
    return pl.pallas_call(
        _node_mid_body,
        grid=(_N // _BN,),
        in_specs=[blk(_BN, _L), blk(_BN, _L), blk(_BN, _L),
                  cst(_L, _L), cst(_L, _L), cst(1, _L), cst(_L, _L),
                  cst(1, _L), cst(_L, _L), cst(_L, _L)],
        out_specs=[blk(_BN, _L)] * 3,
        out_shape=[jax.ShapeDtypeStruct((_N, _L), jnp.float32)] * 3,
    )(v, p0, p1, uv, ua, c1, u2, c2, ws, wr)


def _node_fin_body(v_ref, p0_ref, p1_ref, uv_ref, ua_ref, c1_ref, u2_ref,
                   c2_ref, d1_ref, e1_ref, d2_ref, e2_ref, base_ref, out_ref):
    v = v_ref[...]
    agg = p0_ref[...] + p1_ref[...]
    g = jnp.maximum(_dot(v, uv_ref[...]) + _dot(agg, ua_ref[...]) + c1_ref[...],
                    0.0)
    vo = v + _ln(_dot(g, u2_ref[...]) + c2_ref[...])
    hd = jnp.maximum(_dot(vo, d1_ref[...]) + e1_ref[...], 0.0)
    acc = _dot(hd, d2_ref[...]) + e2_ref[...]
    out_ref[...] = base_ref[...] + acc


def _node_fin(v, p0, p1, uv, ua, c1, u2, c2, d1, e1, d2, e2, base):
    blk = lambda r, c: pl.BlockSpec((r, c), lambda i: (i, 0))
    cst = lambda r, c: pl.BlockSpec((r, c), lambda i: (0, 0))
    return pl.pallas_call(
        _node_fin_body,
        grid=(_N // _BN,),
        in_specs=[blk(_BN, _L), blk(_BN, _L), blk(_BN, _L),
                  cst(_L, _L), cst(_L, _L), cst(1, _L), cst(_L, _L),
                  cst(1, _L), cst(_L, _L), cst(1, _L), cst(_L, _L),
                  cst(1, _L), blk(_BN, _L)],
        out_specs=blk(_BN, _L),
        out_shape=jax.ShapeDtypeStruct((_N, _L), jnp.float32),
    )(v, p0, p1, uv, ua, c1, u2, c2, d1, e1, d2, e2, base)


def kernel(position_sequence, params, particle_types, senders, receivers,
           n_particles_per_example):
    f32 = jnp.float32
    pos = position_sequence
    most = pos[:, -1]

    nvel = (pos[:, 1:] - pos[:, :-1]).reshape(_N, -1)
    dist = jnp.concatenate([most - 0.1, 0.9 - most], axis=1)
    dist = jnp.clip(dist * (1.0 / _RADIUS), -1.0, 1.0)
    onehot = (particle_types[:, None]
              == jnp.arange(_NTYPES, dtype=particle_types.dtype)[None, :])
    x = jnp.concatenate([nvel, dist, onehot.astype(f32),
                         jnp.zeros((_N, 2), f32)], axis=1)

    a1w, a1b = params['enc_node'][0]
    a2w, a2b = params['enc_node'][1]
    wn1 = jnp.concatenate(
        [a1w[:21], params['type_emb'] @ a1w[21:37], jnp.zeros((2, _L), f32)],
        axis=0)
    b1w, b1b = params['enc_edge'][0]
    b2w, b2b = params['enc_edge'][1]
    be1 = jnp.concatenate([b1w[:3], jnp.zeros((13, _L), f32)], axis=0)
    ben = b1w[3:4]
    dw1, db1 = params['dec'][0]
    dw2, db2 = params['dec'][1]
    dw2p = jnp.concatenate([dw2, jnp.zeros((_L, _L - _DIMS), f32)], axis=1)
    db2p = jnp.concatenate([db2, jnp.zeros((_L - _DIMS,), f32)])[None, :]

    steps = []
    for sp in params['proc']:
        w1, bb1 = sp['edge'][0]
        w2, bb2 = sp['edge'][1]
        u1, cc1 = sp['node'][0]
        u2, cc2 = sp['node'][1]
        steps.append(dict(
            w1e=w1[:_L], w1s=w1[_L:2 * _L], w1r=w1[2 * _L:], b1=bb1[None, :],
            w2=w2, b2=bb2[None, :], uv=u1[:_L], ua=u1[_L:], c1=cc1[None, :],
            u2=u2, c2=cc2[None, :]))

    npad = _EPAD - _E
    i32 = jnp.int32
    s_i = senders.astype(i32)
    r_i = receivers.astype(i32)
    sidx = jnp.concatenate([s_i, jnp.zeros((npad,), i32)]).reshape(-1, _IDXW)
    ridx_g = jnp.concatenate([r_i, jnp.zeros((npad,), i32)]).reshape(-1, _IDXW)
    ridx_s = jnp.concatenate([r_i, jnp.full((npad,), _N, i32)]).reshape(-1, _IDXW)

    tpos = jnp.concatenate([most, jnp.zeros((_N, 13), f32)], axis=1)

    spos, rpos = _gather2(16, tpos, tpos, sidx, ridx_g)
    e = _enc_edge(spos, rpos, be1, ben, b1b[None, :], b2w, b2b[None, :])
    v, ps, pr = _enc_node(x, wn1, a1b[None, :], a2w, a2b[None, :],
                          steps[0]['w1s'], steps[0]['w1r'])

    base = jnp.concatenate(
        [2.0 * most - pos[:, -2], jnp.zeros((_N, _L - _DIMS), f32)], axis=1)

    for t in range(_MP_STEPS):
        st = steps[t]
        gs, gr = _gather2(_L, ps, pr, sidx, ridx_g)
        e, en = _edge_step(e, gs, gr, st['w1e'], st['b1'], st['w2'], st['b2'])
        parts = _scatter(en, ridx_s)
        if t < _MP_STEPS - 1:
            nx = steps[t + 1]
            v, ps, pr = _node_mid(v, parts[0], parts[1], st['uv'], st['ua'],
                                  st['c1'], st['u2'], st['c2'],
                                  nx['w1s'], nx['w1r'])
        else:
            out = _node_fin(v, parts[0], parts[1], st['uv'], st['ua'],
                            st['c1'], st['u2'], st['c2'], dw1, db1[None, :],
                            dw2p, db2p, base)

    return out[:, :_DIMS]


# speedup vs baseline: 1.8236x; 1.0023x over previous
import functools

import jax
import jax.numpy as jnp
from jax import lax
from jax.experimental import pallas as pl
from jax.experimental.pallas import tpu as pltpu
from jax.experimental.pallas import tpu_sc as plsc

_N = 10000
_E = 160000
_DIMS = 3
_L = 64
_RADIUS = 0.015
_NTYPES = 9
_MP_STEPS = 5

_NC = 2
_NS = 16
_NW = _NC * _NS
_IDXW = 128
_CPW = 40
_EPAD = _NW * _CPW * _IDXW
_KB = 8
_NBLK = _CPW // _KB
_NACC = _N + 8
_RPS = _N // _NS

_MESH = plsc.VectorSubcoreMesh(core_axis_name="c", subcore_axis_name="s")
_SC_PARAMS = pltpu.CompilerParams(use_tc_tiling_on_sc=False)


_GCH = 4
_GRING = 3
_SRING = 2


_GW = _GCH * _IDXW
_IPW = _CPW * _IDXW


def _gather2_body(ts_ref, tr_ref, si_ref, ri_ref, gs_ref, gr_ref,
                  idx_s, idx_r, bufs, g0, g1, g2, o0, o1, o2):
    gsem = (g0, g1, g2)
    osem = (o0, o1, o2)
    wid = lax.axis_index("s") * _NC + lax.axis_index("c")
    base = wid * _IPW
    pltpu.sync_copy(si_ref.at[pl.ds(base, _IPW)], idx_s)
    pltpu.sync_copy(ri_ref.at[pl.ds(base, _IPW)], idx_r)
    nch = _IPW // _GW
    chunks = ([(idx_s, ts_ref, gs_ref, c) for c in range(nch)]
              + [(idx_r, tr_ref, gr_ref, c) for c in range(nch)])
    gds = [None] * _GRING
    ods = [None] * _GRING
    outinfo = [None] * _GRING
    for ci, (idx_v, tbl, out, c) in enumerate(chunks):
        b = ci % _GRING
        if ods[b] is not None:
            ods[b].wait()
        gds_b = pltpu.async_copy(tbl.at[idx_v.at[pl.ds(c * _GW, _GW)]],
                                 bufs.at[b], gsem[b])
        if ci >= 1:
            pb = (ci - 1) % _GRING
            gds[pb].wait()
            pout, prow = outinfo[pb]
            ods[pb] = pltpu.async_copy(bufs.at[pb],
                                       pout.at[pl.ds(prow, _GW)],
                                       osem[pb])
        gds[b] = gds_b
        outinfo[b] = (out, base + c * _GW)
    lb = (len(chunks) - 1) % _GRING
    gds[lb].wait()
    pout, prow = outinfo[lb]
    ods[lb] = pltpu.async_copy(bufs.at[lb], pout.at[pl.ds(prow, _GW)],
                               osem[lb])
    for b in range(_GRING):
        if ods[b] is not None:
            ods[b].wait()


def _gather2(d, tbl_s, tbl_r, sidx, ridx):
    fn = pl.kernel(
        _gather2_body,
        out_type=[jax.ShapeDtypeStruct((_EPAD, d), jnp.float32)] * 2,
        mesh=_MESH,
        compiler_params=_SC_PARAMS,
        scratch_types=[
            pltpu.VMEM((_IPW,), jnp.int32),
            pltpu.VMEM((_IPW,), jnp.int32),
            pltpu.VMEM((_GRING, _GW, d), jnp.float32),
        ] + [pltpu.SemaphoreType.DMA] * (2 * _GRING),
    )
    return fn(tbl_s, tbl_r, sidx, ridx)


_ZR = 64


def _scatter_body(en_ref, ri_ref, out_ref, idx_v, bufs, zbuf, l0, l1,
                  s0, s1, acc_sh):
    lsem = (l0, l1)
    ssem = (s0, s1)
    cid = lax.axis_index("c")
    sid = lax.axis_index("s")
    wid = sid * _NC + cid
    pltpu.sync_copy(ri_ref.at[pl.ds(wid * _CPW, _CPW)], idx_v)

    def _zb(i, _):
        zbuf[i // 4, pl.ds((i % 4) * 16, 16)] = jnp.zeros((16,), jnp.float32)
        return _

    lax.fori_loop(0, _ZR * 4, _zb, 0)
    nfull = _RPS // _ZR
    for zi in range(nfull):
        pltpu.sync_copy(zbuf, acc_sh.at[pl.ds(sid * _RPS + zi * _ZR, _ZR)])
    rem = _RPS - nfull * _ZR
    if rem:
        pltpu.sync_copy(zbuf.at[pl.ds(0, rem)],
                        acc_sh.at[pl.ds(sid * _RPS + nfull * _ZR, rem)])

    @pl.when(sid == 0)
    def _():
        pltpu.sync_copy(zbuf.at[pl.ds(0, _NACC - _N)],
                        acc_sh.at[pl.ds(_N, _NACC - _N)])

    plsc.subcore_barrier()
    nch = _CPW // _GCH
    lds = [None] * _SRING
    sds = [None] * _SRING
    for c in range(nch):
        b = c % _SRING
        if sds[b] is not None:
            for dsc in sds[b]:
                dsc.wait()
        row0 = wid * _CPW + c * _GCH
        lds[b] = pltpu.async_copy(
            en_ref.at[pl.ds(row0 * _IDXW, _GCH * _IDXW)], bufs.at[b],
            lsem[b])
        if c >= 1:
            pb = (c - 1) % _SRING
            lds[pb].wait()
            sds[pb] = [pltpu.async_copy(bufs.at[pb, pl.ds(j * _IDXW, _IDXW)],
                                        acc_sh.at[idx_v.at[(c - 1) * _GCH + j]],
                                        ssem[pb], add=True)
                       for j in range(_GCH)]
    lb = (nch - 1) % _SRING
    lds[lb].wait()
    sds[lb] = [pltpu.async_copy(bufs.at[lb, pl.ds(j * _IDXW, _IDXW)],
                                acc_sh.at[idx_v.at[(nch - 1) * _GCH + j]],
                                ssem[lb], add=True)
               for j in range(_GCH)]
    for b in range(_SRING):
        if sds[b] is not None:
            for dsc in sds[b]:
                dsc.wait()
    plsc.subcore_barrier()
    pltpu.sync_copy(acc_sh.at[pl.ds(sid * _RPS, _RPS)],
                    out_ref.at[cid, pl.ds(sid * _RPS, _RPS)])


def _scatter(en, ridx):
    fn = pl.kernel(
        _scatter_body,
        out_type=jax.ShapeDtypeStruct((_NC, _N, _L), jnp.float32),
        mesh=_MESH,
        compiler_params=_SC_PARAMS,
        scratch_types=[
            pltpu.VMEM((_CPW, _IDXW), jnp.int32),
            pltpu.VMEM((_SRING, _GCH * _IDXW, _L), jnp.float32),
            pltpu.VMEM((_ZR, _L), jnp.float32),
        ] + [pltpu.SemaphoreType.DMA] * (2 * _SRING) + [
            pltpu.VMEM_SHARED((_NACC, _L), jnp.float32),
        ],
    )
    return fn(en, ridx)


_BE = 2048
_BN = 2000


def _ln(x):
    m = jnp.mean(x, axis=-1, keepdims=True)
    xc = x - m
    var = jnp.mean(xc * xc, axis=-1, keepdims=True)
    return xc * lax.rsqrt(var + 1e-6)


def _dot(a, b):
    return jnp.dot(a, b, preferred_element_type=jnp.float32)


def _edge_body(e_ref, gs_ref, gr_ref, w1_ref, b1_ref, w2_ref, b2_ref,
               eo_ref, en_ref):
    pre = _dot(e_ref[...], w1_ref[...]) + gs_ref[...] + gr_ref[...] + b1_ref[...]
    h = jnp.maximum(pre, 0.0)
    en = _ln(_dot(h, w2_ref[...]) + b2_ref[...])
    en_ref[...] = en
    eo_ref[...] = e_ref[...] + en


def _edge_step(e, gs, gr, w1, b1, w2, b2):
    blk = lambda r, c: pl.BlockSpec((r, c), lambda i: (i, 0))
    cst = lambda r, c: pl.BlockSpec((r, c), lambda i: (0, 0))
    return pl.pallas_call(
        _edge_body,
        grid=(_EPAD // _BE,),
        in_specs=[blk(_BE, _L), blk(_BE, _L), blk(_BE, _L),
                  cst(_L, _L), cst(1, _L), cst(_L, _L), cst(1, _L)],
        out_specs=[blk(_BE, _L), blk(_BE, _L)],
        out_shape=[jax.ShapeDtypeStruct((_EPAD, _L), jnp.float32)] * 2,
    )(e, gs, gr, w1, b1, w2, b2)


def _enc_edge_body(sp_ref, rp_ref, w1_ref, wn_ref, b1_ref, w2_ref, b2_ref,
                   e_ref):
    rel = (sp_ref[...] - rp_ref[...]) * (1.0 / _RADIUS)
    nrm = jnp.sqrt(jnp.sum(rel * rel, axis=-1, keepdims=True))
    pre = _dot(rel, w1_ref[...]) + nrm * wn_ref[...] + b1_ref[...]
    h = jnp.maximum(pre, 0.0)
    e_ref[...] = _ln(_dot(h, w2_ref[...]) + b2_ref[...])


def _enc_edge(spos, rpos, w1p, wn, b1, w2, b2):
    blk = lambda r, c: pl.BlockSpec((r, c), lambda i: (i, 0))
    cst = lambda r, c: pl.BlockSpec((r, c), lambda i: (0, 0))
    return pl.pallas_call(
        _enc_edge_body,
        grid=(_EPAD // _BE,),
        in_specs=[blk(_BE, 16), blk(_BE, 16),
                  cst(16, _L), cst(1, _L), cst(1, _L), cst(_L, _L), cst(1, _L)],
        out_specs=blk(_BE, _L),
        out_shape=jax.ShapeDtypeStruct((_EPAD, _L), jnp.float32),
    )(spos, rpos, w1p, wn, b1, w2, b2)


def _enc_node_body(x_ref, w1_ref, b1_ref, w2_ref, b2_ref, ws_ref, wr_ref,
                   v_ref, ps_ref, pr_ref):
    h = jnp.maximum(_dot(x_ref[...], w1_ref[...]) + b1_ref[...], 0.0)
    v = _ln(_dot(h, w2_ref[...]) + b2_ref[...])
    v_ref[...] = v
    ps_ref[...] = _dot(v, ws_ref[...])
    pr_ref[...] = _dot(v, wr_ref[...])


def _enc_node(x, w1, b1, w2, b2, ws, wr):
    blk = lambda r, c: pl.BlockSpec((r, c), lambda i: (i, 0))
    cst = lambda r, c: pl.BlockSpec((r, c), lambda i: (0, 0))
    return pl.pallas_call(
        _enc_node_body,
        grid=(_N // _BN,),
        in_specs=[blk(_BN, 32), cst(32, _L), cst(1, _L), cst(_L, _L),
                  cst(1, _L), cst(_L, _L), cst(_L, _L)],
        out_specs=[blk(_BN, _L)] * 3,
        out_shape=[jax.ShapeDtypeStruct((_N, _L), jnp.float32)] * 3,
    )(x, w1, b1, w2, b2, ws, wr)


def _node_mid_body(v_ref, p0_ref, p1_ref, uv_ref, ua_ref, c1_ref, u2_ref,
                   c2_ref, ws_ref, wr_ref, vo_ref, ps_ref, pr_ref):
    v = v_ref[...]
    agg = p0_ref[...] + p1_ref[...]
    g = jnp.maximum(_dot(v, uv_ref[...]) + _dot(agg, ua_ref[...]) + c1_ref[...],
                    0.0)
    vo = v + _ln(_dot(g, u2_ref[...]) + c2_ref[...])
    vo_ref[...] = vo
    ps_ref[...] = _dot(vo, ws_ref[...])
    pr_ref[...] = _dot(vo, wr_ref[...])


def _node_mid(v, p0, p1, uv, ua, c1, u2, c2, ws, wr):
    blk = lambda r, c: pl.BlockSpec((r, c), lambda i: (i, 0))
    cst = lambda r, c: pl.BlockSpec((r, c), lambda i: (0, 0))
    return pl.pallas_call(
        _node_mid_body,
        grid=(_N // _BN,),
        in_specs=[blk(_BN, _L), blk(_BN, _L), blk(_BN, _L),
                  cst(_L, _L), cst(_L, _L), cst(1, _L), cst(_L, _L),
                  cst(1, _L), cst(_L, _L), cst(_L, _L)],
        out_specs=[blk(_BN, _L)] * 3,
        out_shape=[jax.ShapeDtypeStruct((_N, _L), jnp.float32)] * 3,
    )(v, p0, p1, uv, ua, c1, u2, c2, ws, wr)


def _node_fin_body(v_ref, p0_ref, p1_ref, uv_ref, ua_ref, c1_ref, u2_ref,
                   c2_ref, d1_ref, e1_ref, d2_ref, e2_ref, base_ref, out_ref):
    v = v_ref[...]
    agg = p0_ref[...] + p1_ref[...]
    g = jnp.maximum(_dot(v, uv_ref[...]) + _dot(agg, ua_ref[...]) + c1_ref[...],
                    0.0)
    vo = v + _ln(_dot(g, u2_ref[...]) + c2_ref[...])
    hd = jnp.maximum(_dot(vo, d1_ref[...]) + e1_ref[...], 0.0)
    acc = _dot(hd, d2_ref[...]) + e2_ref[...]
    out_ref[...] = base_ref[...] + acc


def _node_fin(v, p0, p1, uv, ua, c1, u2, c2, d1, e1, d2, e2, base):
    blk = lambda r, c: pl.BlockSpec((r, c), lambda i: (i, 0))
    cst = lambda r, c: pl.BlockSpec((r, c), lambda i: (0, 0))
    return pl.pallas_call(
        _node_fin_body,
        grid=(_N // _BN,),
        in_specs=[blk(_BN, _L), blk(_BN, _L), blk(_BN, _L),
                  cst(_L, _L), cst(_L, _L), cst(1, _L), cst(_L, _L),
                  cst(1, _L), cst(_L, _L), cst(1, _L), cst(_L, _L),
                  cst(1, _L), blk(_BN, _L)],
        out_specs=blk(_BN, _L),
        out_shape=jax.ShapeDtypeStruct((_N, _L), jnp.float32),
    )(v, p0, p1, uv, ua, c1, u2, c2, d1, e1, d2, e2, base)


def kernel(position_sequence, params, particle_types, senders, receivers,
           n_particles_per_example):
    f32 = jnp.float32
    pos = position_sequence
    most = pos[:, -1]

    nvel = (pos[:, 1:] - pos[:, :-1]).reshape(_N, -1)
    dist = jnp.concatenate([most - 0.1, 0.9 - most], axis=1)
    dist = jnp.clip(dist * (1.0 / _RADIUS), -1.0, 1.0)
    onehot = (particle_types[:, None]
              == jnp.arange(_NTYPES, dtype=particle_types.dtype)[None, :])
    x = jnp.concatenate([nvel, dist, onehot.astype(f32),
                         jnp.zeros((_N, 2), f32)], axis=1)

    a1w, a1b = params['enc_node'][0]
    a2w, a2b = params['enc_node'][1]
    wn1 = jnp.concatenate(
        [a1w[:21], params['type_emb'] @ a1w[21:37], jnp.zeros((2, _L), f32)],
        axis=0)
    b1w, b1b = params['enc_edge'][0]
    b2w, b2b = params['enc_edge'][1]
    be1 = jnp.concatenate([b1w[:3], jnp.zeros((13, _L), f32)], axis=0)
    ben = b1w[3:4]
    dw1, db1 = params['dec'][0]
    dw2, db2 = params['dec'][1]
    dw2p = jnp.concatenate([dw2, jnp.zeros((_L, _L - _DIMS), f32)], axis=1)
    db2p = jnp.concatenate([db2, jnp.zeros((_L - _DIMS,), f32)])[None, :]

    steps = []
    for sp in params['proc']:
        w1, bb1 = sp['edge'][0]
        w2, bb2 = sp['edge'][1]
        u1, cc1 = sp['node'][0]
        u2, cc2 = sp['node'][1]
        steps.append(dict(
            w1e=w1[:_L], w1s=w1[_L:2 * _L], w1r=w1[2 * _L:], b1=bb1[None, :],
            w2=w2, b2=bb2[None, :], uv=u1[:_L], ua=u1[_L:], c1=cc1[None, :],
            u2=u2, c2=cc2[None, :]))

    npad = _EPAD - _E
    i32 = jnp.int32
    s_i = senders.astype(i32)
    r_i = receivers.astype(i32)
    sidx = jnp.concatenate([s_i, jnp.zeros((npad,), i32)])
    ridx_g = jnp.concatenate([r_i, jnp.zeros((npad,), i32)])
    ridx_s = jnp.concatenate([r_i, jnp.full((npad,), _N, i32)]).reshape(-1, _IDXW)

    tpos = jnp.concatenate([most, jnp.zeros((_N, 13), f32)], axis=1)

    spos, rpos = _gather2(16, tpos, tpos, sidx, ridx_g)
    e = _enc_edge(spos, rpos, be1, ben, b1b[None, :], b2w, b2b[None, :])
    v, ps, pr = _enc_node(x, wn1, a1b[None, :], a2w, a2b[None, :],
                          steps[0]['w1s'], steps[0]['w1r'])

    base = jnp.concatenate(
        [2.0 * most - pos[:, -2], jnp.zeros((_N, _L - _DIMS), f32)], axis=1)

    for t in range(_MP_STEPS):
        st = steps[t]
        gs, gr = _gather2(_L, ps, pr, sidx, ridx_g)
        e, en = _edge_step(e, gs, gr, st['w1e'], st['b1'], st['w2'], st['b2'])
        parts = _scatter(en, ridx_s)
        if t < _MP_STEPS - 1:
            nx = steps[t + 1]
            v, ps, pr = _node_mid(v, parts[0], parts[1], st['uv'], st['ua'],
                                  st['c1'], st['u2'], st['c2'],
                                  nx['w1s'], nx['w1r'])
        else:
            out = _node_fin(v, parts[0], parts[1], st['uv'], st['ua'],
                            st['c1'], st['u2'], st['c2'], dw1, db1[None, :],
                            dw2p, db2p, base)

    return out[:, :_DIMS]


# speedup vs baseline: 1.9088x; 1.0467x over previous
import functools

import jax
import jax.numpy as jnp
from jax import lax
from jax.experimental import pallas as pl
from jax.experimental.pallas import tpu as pltpu
from jax.experimental.pallas import tpu_sc as plsc

_N = 10000
_E = 160000
_DIMS = 3
_L = 64
_RADIUS = 0.015
_NTYPES = 9
_MP_STEPS = 5

_NC = 2
_NS = 16
_NW = _NC * _NS
_IDXW = 128
_CPW = 40
_EPAD = _NW * _CPW * _IDXW
_KB = 8
_NBLK = _CPW // _KB
_NACC = _N + 8
_RPS = _N // _NS

_MESH = plsc.VectorSubcoreMesh(core_axis_name="c", subcore_axis_name="s")
_SC_PARAMS = pltpu.CompilerParams(use_tc_tiling_on_sc=False)


_GCH = 4
_GRING = 3
_SRING = 2


_GW = _GCH * _IDXW
_IPW = _CPW * _IDXW


def _gather2_body(ts_ref, tr_ref, si_ref, ri_ref, gs_ref, gr_ref,
                  idx_s, idx_r, bufs, g0, g1, g2, o0, o1, o2):
    gsem = (g0, g1, g2)
    osem = (o0, o1, o2)
    wid = lax.axis_index("s") * _NC + lax.axis_index("c")
    base = wid * _IPW
    pltpu.sync_copy(si_ref.at[pl.ds(base, _IPW)], idx_s)
    pltpu.sync_copy(ri_ref.at[pl.ds(base, _IPW)], idx_r)
    nch = _IPW // _GW
    chunks = ([(idx_s, ts_ref, gs_ref, c) for c in range(nch)]
              + [(idx_r, tr_ref, gr_ref, c) for c in range(nch)])
    gds = [None] * _GRING
    ods = [None] * _GRING
    outinfo = [None] * _GRING
    for ci, (idx_v, tbl, out, c) in enumerate(chunks):
        b = ci % _GRING
        if ods[b] is not None:
            ods[b].wait()
        gds_b = pltpu.async_copy(tbl.at[idx_v.at[pl.ds(c * _GW, _GW)]],
                                 bufs.at[b], gsem[b])
        if ci >= 1:
            pb = (ci - 1) % _GRING
            gds[pb].wait()
            pout, prow = outinfo[pb]
            ods[pb] = pltpu.async_copy(bufs.at[pb],
                                       pout.at[pl.ds(prow, _GW)],
                                       osem[pb])
        gds[b] = gds_b
        outinfo[b] = (out, base + c * _GW)
    lb = (len(chunks) - 1) % _GRING
    gds[lb].wait()
    pout, prow = outinfo[lb]
    ods[lb] = pltpu.async_copy(bufs.at[lb], pout.at[pl.ds(prow, _GW)],
                               osem[lb])
    for b in range(_GRING):
        if ods[b] is not None:
            ods[b].wait()


def _gather2(d, tbl_s, tbl_r, sidx, ridx):
    dt = tbl_s.dtype
    fn = pl.kernel(
        _gather2_body,
        out_type=[jax.ShapeDtypeStruct((_EPAD, d), dt)] * 2,
        mesh=_MESH,
        compiler_params=_SC_PARAMS,
        scratch_types=[
            pltpu.VMEM((_IPW,), jnp.int32),
            pltpu.VMEM((_IPW,), jnp.int32),
            pltpu.VMEM((_GRING, _GW, d), dt),
        ] + [pltpu.SemaphoreType.DMA] * (2 * _GRING),
    )
    return fn(tbl_s, tbl_r, sidx, ridx)


_ZR = 64


def _scatter_body(en_ref, ri_ref, out_ref, idx_v, bufs, zbuf, l0, l1,
                  s0, s1, acc_sh):
    lsem = (l0, l1)
    ssem = (s0, s1)
    cid = lax.axis_index("c")
    sid = lax.axis_index("s")
    wid = sid * _NC + cid
    pltpu.sync_copy(ri_ref.at[pl.ds(wid * _CPW, _CPW)], idx_v)

    def _zb(i, _):
        zbuf[i // 4, pl.ds((i % 4) * 16, 16)] = jnp.zeros((16,), jnp.float32)
        return _

    lax.fori_loop(0, _ZR * 4, _zb, 0)
    nfull = _RPS // _ZR
    for zi in range(nfull):
        pltpu.sync_copy(zbuf, acc_sh.at[pl.ds(sid * _RPS + zi * _ZR, _ZR)])
    rem = _RPS - nfull * _ZR
    if rem:
        pltpu.sync_copy(zbuf.at[pl.ds(0, rem)],
                        acc_sh.at[pl.ds(sid * _RPS + nfull * _ZR, rem)])

    @pl.when(sid == 0)
    def _():
        pltpu.sync_copy(zbuf.at[pl.ds(0, _NACC - _N)],
                        acc_sh.at[pl.ds(_N, _NACC - _N)])

    plsc.subcore_barrier()
    nch = _CPW // _GCH
    lds = [None] * _SRING
    sds = [None] * _SRING
    for c in range(nch):
        b = c % _SRING
        if sds[b] is not None:
            for dsc in sds[b]:
                dsc.wait()
        row0 = wid * _CPW + c * _GCH
        lds[b] = pltpu.async_copy(
            en_ref.at[pl.ds(row0 * _IDXW, _GCH * _IDXW)], bufs.at[b],
            lsem[b])
        if c >= 1:
            pb = (c - 1) % _SRING
            lds[pb].wait()
            sds[pb] = [pltpu.async_copy(bufs.at[pb, pl.ds(j * _IDXW, _IDXW)],
                                        acc_sh.at[idx_v.at[(c - 1) * _GCH + j]],
                                        ssem[pb], add=True)
                       for j in range(_GCH)]
    lb = (nch - 1) % _SRING
    lds[lb].wait()
    sds[lb] = [pltpu.async_copy(bufs.at[lb, pl.ds(j * _IDXW, _IDXW)],
                                acc_sh.at[idx_v.at[(nch - 1) * _GCH + j]],
                                ssem[lb], add=True)
               for j in range(_GCH)]
    for b in range(_SRING):
        if sds[b] is not None:
            for dsc in sds[b]:
                dsc.wait()
    plsc.subcore_barrier()
    pltpu.sync_copy(acc_sh.at[pl.ds(sid * _RPS, _RPS)],
                    out_ref.at[cid, pl.ds(sid * _RPS, _RPS)])


def _scatter(en, ridx):
    fn = pl.kernel(
        _scatter_body,
        out_type=jax.ShapeDtypeStruct((_NC, _N, _L), jnp.float32),
        mesh=_MESH,
        compiler_params=_SC_PARAMS,
        scratch_types=[
            pltpu.VMEM((_CPW, _IDXW), jnp.int32),
            pltpu.VMEM((_SRING, _GCH * _IDXW, _L), jnp.float32),
            pltpu.VMEM((_ZR, _L), jnp.float32),
        ] + [pltpu.SemaphoreType.DMA] * (2 * _SRING) + [
            pltpu.VMEM_SHARED((_NACC, _L), jnp.float32),
        ],
    )
    return fn(en, ridx)


_BE = 2048
_BN = 2000


def _ln(x):
    m = jnp.mean(x, axis=-1, keepdims=True)
    xc = x - m
    var = jnp.mean(xc * xc, axis=-1, keepdims=True)
    return xc * lax.rsqrt(var + 1e-6)


def _dot(a, b):
    return jnp.dot(a, b, preferred_element_type=jnp.float32)


def _edge_body(e_ref, gs_ref, gr_ref, w1_ref, b1_ref, w2_ref, b2_ref,
               eo_ref, en_ref):
    pre = (_dot(e_ref[...], w1_ref[...])
           + gs_ref[...].astype(jnp.float32)
           + gr_ref[...].astype(jnp.float32) + b1_ref[...])
    h = jnp.maximum(pre, 0.0)
    en = _ln(_dot(h, w2_ref[...]) + b2_ref[...])
    en_ref[...] = en
    eo_ref[...] = e_ref[...] + en


def _edge_step(e, gs, gr, w1, b1, w2, b2):
    blk = lambda r, c: pl.BlockSpec((r, c), lambda i: (i, 0))
    cst = lambda r, c: pl.BlockSpec((r, c), lambda i: (0, 0))
    return pl.pallas_call(
        _edge_body,
        grid=(_EPAD // _BE,),
        in_specs=[blk(_BE, _L), blk(_BE, _L), blk(_BE, _L),
                  cst(_L, _L), cst(1, _L), cst(_L, _L), cst(1, _L)],
        out_specs=[blk(_BE, _L), blk(_BE, _L)],
        out_shape=[jax.ShapeDtypeStruct((_EPAD, _L), jnp.float32)] * 2,
    )(e, gs, gr, w1, b1, w2, b2)


def _enc_edge_body(sp_ref, rp_ref, w1_ref, wn_ref, b1_ref, w2_ref, b2_ref,
                   e_ref):
    rel = (sp_ref[...] - rp_ref[...]) * (1.0 / _RADIUS)
    nrm = jnp.sqrt(jnp.sum(rel * rel, axis=-1, keepdims=True))
    pre = _dot(rel, w1_ref[...]) + nrm * wn_ref[...] + b1_ref[...]
    h = jnp.maximum(pre, 0.0)
    e_ref[...] = _ln(_dot(h, w2_ref[...]) + b2_ref[...])


def _enc_edge(spos, rpos, w1p, wn, b1, w2, b2):
    blk = lambda r, c: pl.BlockSpec((r, c), lambda i: (i, 0))
    cst = lambda r, c: pl.BlockSpec((r, c), lambda i: (0, 0))
    return pl.pallas_call(
        _enc_edge_body,
        grid=(_EPAD // _BE,),
        in_specs=[blk(_BE, 16), blk(_BE, 16),
                  cst(16, _L), cst(1, _L), cst(1, _L), cst(_L, _L), cst(1, _L)],
        out_specs=blk(_BE, _L),
        out_shape=jax.ShapeDtypeStruct((_EPAD, _L), jnp.float32),
    )(spos, rpos, w1p, wn, b1, w2, b2)


def _enc_node_body(x_ref, w1_ref, b1_ref, w2_ref, b2_ref, ws_ref, wr_ref,
                   v_ref, ps_ref, pr_ref):
    h = jnp.maximum(_dot(x_ref[...], w1_ref[...]) + b1_ref[...], 0.0)
    v = _ln(_dot(h, w2_ref[...]) + b2_ref[...])
    v_ref[...] = v
    ps_ref[...] = _dot(v, ws_ref[...]).astype(jnp.bfloat16)
    pr_ref[...] = _dot(v, wr_ref[...]).astype(jnp.bfloat16)


def _enc_node(x, w1, b1, w2, b2, ws, wr):
    blk = lambda r, c: pl.BlockSpec((r, c), lambda i: (i, 0))
    cst = lambda r, c: pl.BlockSpec((r, c), lambda i: (0, 0))
    return pl.pallas_call(
        _enc_node_body,
        grid=(_N // _BN,),
        in_specs=[blk(_BN, 32), cst(32, _L), cst(1, _L), cst(_L, _L),
                  cst(1, _L), cst(_L, _L), cst(_L, _L)],
        out_specs=[blk(_BN, _L)] * 3,
        out_shape=[jax.ShapeDtypeStruct((_N, _L), jnp.float32),
                   jax.ShapeDtypeStruct((_N, _L), jnp.bfloat16),
                   jax.ShapeDtypeStruct((_N, _L), jnp.bfloat16)],
    )(x, w1, b1, w2, b2, ws, wr)


def _node_mid_body(v_ref, p0_ref, p1_ref, uv_ref, ua_ref, c1_ref, u2_ref,
                   c2_ref, ws_ref, wr_ref, vo_ref, ps_ref, pr_ref):
    v = v_ref[...]
    agg = p0_ref[...] + p1_ref[...]
    g = jnp.maximum(_dot(v, uv_ref[...]) + _dot(agg, ua_ref[...]) + c1_ref[...],
                    0.0)
    vo = v + _ln(_dot(g, u2_ref[...]) + c2_ref[...])
    vo_ref[...] = vo
    ps_ref[...] = _dot(vo, ws_ref[...]).astype(jnp.bfloat16)
    pr_ref[...] = _dot(vo, wr_ref[...]).astype(jnp.bfloat16)


def _node_mid(v, p0, p1, uv, ua, c1, u2, c2, ws, wr):
    blk = lambda r, c: pl.BlockSpec((r, c), lambda i: (i, 0))
    cst = lambda r, c: pl.BlockSpec((r, c), lambda i: (0, 0))
    return pl.pallas_call(
        _node_mid_body,
        grid=(_N // _BN,),
        in_specs=[blk(_BN, _L), blk(_BN, _L), blk(_BN, _L),
                  cst(_L, _L), cst(_L, _L), cst(1, _L), cst(_L, _L),
                  cst(1, _L), cst(_L, _L), cst(_L, _L)],
        out_specs=[blk(_BN, _L)] * 3,
        out_shape=[jax.ShapeDtypeStruct((_N, _L), jnp.float32),
                   jax.ShapeDtypeStruct((_N, _L), jnp.bfloat16),
                   jax.ShapeDtypeStruct((_N, _L), jnp.bfloat16)],
    )(v, p0, p1, uv, ua, c1, u2, c2, ws, wr)


def _node_fin_body(v_ref, p0_ref, p1_ref, uv_ref, ua_ref, c1_ref, u2_ref,
                   c2_ref, d1_ref, e1_ref, d2_ref, e2_ref, base_ref, out_ref):
    v = v_ref[...]
    agg = p0_ref[...] + p1_ref[...]
    g = jnp.maximum(_dot(v, uv_ref[...]) + _dot(agg, ua_ref[...]) + c1_ref[...],
                    0.0)
    vo = v + _ln(_dot(g, u2_ref[...]) + c2_ref[...])
    hd = jnp.maximum(_dot(vo, d1_ref[...]) + e1_ref[...], 0.0)
    acc = _dot(hd, d2_ref[...]) + e2_ref[...]
    out_ref[...] = base_ref[...] + acc


def _node_fin(v, p0, p1, uv, ua, c1, u2, c2, d1, e1, d2, e2, base):
    blk = lambda r, c: pl.BlockSpec((r, c), lambda i: (i, 0))
    cst = lambda r, c: pl.BlockSpec((r, c), lambda i: (0, 0))
    return pl.pallas_call(
        _node_fin_body,
        grid=(_N // _BN,),
        in_specs=[blk(_BN, _L), blk(_BN, _L), blk(_BN, _L),
                  cst(_L, _L), cst(_L, _L), cst(1, _L), cst(_L, _L),
                  cst(1, _L), cst(_L, _L), cst(1, _L), cst(_L, _L),
                  cst(1, _L), blk(_BN, _L)],
        out_specs=blk(_BN, _L),
        out_shape=jax.ShapeDtypeStruct((_N, _L), jnp.float32),
    )(v, p0, p1, uv, ua, c1, u2, c2, d1, e1, d2, e2, base)


def kernel(position_sequence, params, particle_types, senders, receivers,
           n_particles_per_example):
    f32 = jnp.float32
    pos = position_sequence
    most = pos[:, -1]

    nvel = (pos[:, 1:] - pos[:, :-1]).reshape(_N, -1)
    dist = jnp.concatenate([most - 0.1, 0.9 - most], axis=1)
    dist = jnp.clip(dist * (1.0 / _RADIUS), -1.0, 1.0)
    onehot = (particle_types[:, None]
              == jnp.arange(_NTYPES, dtype=particle_types.dtype)[None, :])
    x = jnp.concatenate([nvel, dist, onehot.astype(f32),
                         jnp.zeros((_N, 2), f32)], axis=1)

    a1w, a1b = params['enc_node'][0]
    a2w, a2b = params['enc_node'][1]
    wn1 = jnp.concatenate(
        [a1w[:21], params['type_emb'] @ a1w[21:37], jnp.zeros((2, _L), f32)],
        axis=0)
    b1w, b1b = params['enc_edge'][0]
    b2w, b2b = params['enc_edge'][1]
    be1 = jnp.concatenate([b1w[:3], jnp.zeros((13, _L), f32)], axis=0)
    ben = b1w[3:4]
    dw1, db1 = params['dec'][0]
    dw2, db2 = params['dec'][1]
    dw2p = jnp.concatenate([dw2, jnp.zeros((_L, _L - _DIMS), f32)], axis=1)
    db2p = jnp.concatenate([db2, jnp.zeros((_L - _DIMS,), f32)])[None, :]

    steps = []
    for sp in params['proc']:
        w1, bb1 = sp['edge'][0]
        w2, bb2 = sp['edge'][1]
        u1, cc1 = sp['node'][0]
        u2, cc2 = sp['node'][1]
        steps.append(dict(
            w1e=w1[:_L], w1s=w1[_L:2 * _L], w1r=w1[2 * _L:], b1=bb1[None, :],
            w2=w2, b2=bb2[None, :], uv=u1[:_L], ua=u1[_L:], c1=cc1[None, :],
            u2=u2, c2=cc2[None, :]))

    npad = _EPAD - _E
    i32 = jnp.int32
    s_i = senders.astype(i32)
    r_i = receivers.astype(i32)
    sidx = jnp.concatenate([s_i, jnp.zeros((npad,), i32)])
    ridx_g = jnp.concatenate([r_i, jnp.zeros((npad,), i32)])
    ridx_s = jnp.concatenate([r_i, jnp.full((npad,), _N, i32)]).reshape(-1, _IDXW)

    tpos = jnp.concatenate([most, jnp.zeros((_N, 13), f32)], axis=1)

    spos, rpos = _gather2(16, tpos, tpos, sidx, ridx_g)
    e = _enc_edge(spos, rpos, be1, ben, b1b[None, :], b2w, b2b[None, :])
    v, ps, pr = _enc_node(x, wn1, a1b[None, :], a2w, a2b[None, :],
                          steps[0]['w1s'], steps[0]['w1r'])

    base = jnp.concatenate(
        [2.0 * most - pos[:, -2], jnp.zeros((_N, _L - _DIMS), f32)], axis=1)

    for t in range(_MP_STEPS):
        st = steps[t]
        gs, gr = _gather2(_L, ps, pr, sidx, ridx_g)
        e, en = _edge_step(e, gs, gr, st['w1e'], st['b1'], st['w2'], st['b2'])
        parts = _scatter(en, ridx_s)
        if t < _MP_STEPS - 1:
            nx = steps[t + 1]
            v, ps, pr = _node_mid(v, parts[0], parts[1], st['uv'], st['ua'],
                                  st['c1'], st['u2'], st['c2'],
                                  nx['w1s'], nx['w1r'])
        else:
            out = _node_fin(v, parts[0], parts[1], st['uv'], st['ua'],
                            st['c1'], st['u2'], st['c2'], dw1, db1[None, :],
                            dw2p, db2p, base)

    return out[:, :_DIMS]


# speedup vs baseline: 1.9858x; 1.0403x over previous
import functools

import jax
import jax.numpy as jnp
from jax import lax
from jax.experimental import pallas as pl
from jax.experimental.pallas import tpu as pltpu
from jax.experimental.pallas import tpu_sc as plsc

_N = 10000
_E = 160000
_DIMS = 3
_L = 64
_RADIUS = 0.015
_NTYPES = 9
_MP_STEPS = 5

_NC = 2
_NS = 16
_NW = _NC * _NS
_IDXW = 128
_CPW = 40
_EPAD = _NW * _CPW * _IDXW
_KB = 8
_NBLK = _CPW // _KB
_NACC = _N + 8
_RPS = _N // _NS

_MESH = plsc.VectorSubcoreMesh(core_axis_name="c", subcore_axis_name="s")
_SC_PARAMS = pltpu.CompilerParams(use_tc_tiling_on_sc=False)


_GCH = 4
_GRING = 3
_SRING = 2


_GW = _GCH * _IDXW
_IPW = _CPW * _IDXW


def _gather2_body(ts_ref, tr_ref, si_ref, ri_ref, gs_ref, gr_ref,
                  idx_s, idx_r, bufs, g0, g1, g2, o0, o1, o2):
    gsem = (g0, g1, g2)
    osem = (o0, o1, o2)
    wid = lax.axis_index("s") * _NC + lax.axis_index("c")
    base = wid * _IPW
    pltpu.sync_copy(si_ref.at[pl.ds(base, _IPW)], idx_s)
    pltpu.sync_copy(ri_ref.at[pl.ds(base, _IPW)], idx_r)
    nch = _IPW // _GW
    chunks = ([(idx_s, ts_ref, gs_ref, c) for c in range(nch)]
              + [(idx_r, tr_ref, gr_ref, c) for c in range(nch)])
    gds = [None] * _GRING
    ods = [None] * _GRING
    outinfo = [None] * _GRING
    for ci, (idx_v, tbl, out, c) in enumerate(chunks):
        b = ci % _GRING
        if ods[b] is not None:
            ods[b].wait()
        gds_b = pltpu.async_copy(tbl.at[idx_v.at[pl.ds(c * _GW, _GW)]],
                                 bufs.at[b], gsem[b])
        if ci >= 1:
            pb = (ci - 1) % _GRING
            gds[pb].wait()
            pout, prow = outinfo[pb]
            ods[pb] = pltpu.async_copy(bufs.at[pb],
                                       pout.at[pl.ds(prow, _GW)],
                                       osem[pb])
        gds[b] = gds_b
        outinfo[b] = (out, base + c * _GW)
    lb = (len(chunks) - 1) % _GRING
    gds[lb].wait()
    pout, prow = outinfo[lb]
    ods[lb] = pltpu.async_copy(bufs.at[lb], pout.at[pl.ds(prow, _GW)],
                               osem[lb])
    for b in range(_GRING):
        if ods[b] is not None:
            ods[b].wait()


def _gather2(d, tbl_s, tbl_r, sidx, ridx):
    dt = tbl_s.dtype
    fn = pl.kernel(
        _gather2_body,
        out_type=[jax.ShapeDtypeStruct((_EPAD, d), dt)] * 2,
        mesh=_MESH,
        compiler_params=_SC_PARAMS,
        scratch_types=[
            pltpu.VMEM((_IPW,), jnp.int32),
            pltpu.VMEM((_IPW,), jnp.int32),
            pltpu.VMEM((_GRING, _GW, d), dt),
        ] + [pltpu.SemaphoreType.DMA] * (2 * _GRING),
    )
    return fn(tbl_s, tbl_r, sidx, ridx)


_ZR = 64


def _scatter_body(en_ref, ri_ref, out_ref, idx_v, bufs, zbuf, l0, l1,
                  s0, s1, acc_sh):
    lsem = (l0, l1)
    ssem = (s0, s1)
    cid = lax.axis_index("c")
    sid = lax.axis_index("s")
    wid = sid * _NC + cid
    pltpu.sync_copy(ri_ref.at[pl.ds(wid * _CPW, _CPW)], idx_v)

    def _zb(i, _):
        zbuf[i // 4, pl.ds((i % 4) * 16, 16)] = jnp.zeros((16,), jnp.float32)
        return _

    lax.fori_loop(0, _ZR * 4, _zb, 0)
    nfull = _RPS // _ZR
    for zi in range(nfull):
        pltpu.sync_copy(zbuf, acc_sh.at[pl.ds(sid * _RPS + zi * _ZR, _ZR)])
    rem = _RPS - nfull * _ZR
    if rem:
        pltpu.sync_copy(zbuf.at[pl.ds(0, rem)],
                        acc_sh.at[pl.ds(sid * _RPS + nfull * _ZR, rem)])

    @pl.when(sid == 0)
    def _():
        pltpu.sync_copy(zbuf.at[pl.ds(0, _NACC - _N)],
                        acc_sh.at[pl.ds(_N, _NACC - _N)])

    plsc.subcore_barrier()
    nch = _CPW // _GCH
    lds = [None] * _SRING
    sds = [None] * _SRING
    for c in range(nch):
        b = c % _SRING
        if sds[b] is not None:
            for dsc in sds[b]:
                dsc.wait()
        row0 = wid * _CPW + c * _GCH
        lds[b] = pltpu.async_copy(
            en_ref.at[pl.ds(row0 * _IDXW, _GCH * _IDXW)], bufs.at[b],
            lsem[b])
        if c >= 1:
            pb = (c - 1) % _SRING
            lds[pb].wait()
            sds[pb] = [pltpu.async_copy(bufs.at[pb, pl.ds(j * _IDXW, _IDXW)],
                                        acc_sh.at[idx_v.at[(c - 1) * _GCH + j]],
                                        ssem[pb], add=True)
                       for j in range(_GCH)]
    lb = (nch - 1) % _SRING
    lds[lb].wait()
    sds[lb] = [pltpu.async_copy(bufs.at[lb, pl.ds(j * _IDXW, _IDXW)],
                                acc_sh.at[idx_v.at[(nch - 1) * _GCH + j]],
                                ssem[lb], add=True)
               for j in range(_GCH)]
    for b in range(_SRING):
        if sds[b] is not None:
            for dsc in sds[b]:
                dsc.wait()
    plsc.subcore_barrier()
    pltpu.sync_copy(acc_sh.at[pl.ds(sid * _RPS, _RPS)],
                    out_ref.at[cid, pl.ds(sid * _RPS, _RPS)])


def _scatter(en, ridx):
    fn = pl.kernel(
        _scatter_body,
        out_type=jax.ShapeDtypeStruct((_NC, _N, _L), jnp.float32),
        mesh=_MESH,
        compiler_params=_SC_PARAMS,
        scratch_types=[
            pltpu.VMEM((_CPW, _IDXW), jnp.int32),
            pltpu.VMEM((_SRING, _GCH * _IDXW, _L), jnp.float32),
            pltpu.VMEM((_ZR, _L), jnp.float32),
        ] + [pltpu.SemaphoreType.DMA] * (2 * _SRING) + [
            pltpu.VMEM_SHARED((_NACC, _L), jnp.float32),
        ],
    )
    return fn(en, ridx)


_BE = 4096
_BN = 2000


def _ln(x):
    m = jnp.mean(x, axis=-1, keepdims=True)
    xc = x - m
    var = jnp.mean(xc * xc, axis=-1, keepdims=True)
    return xc * lax.rsqrt(var + 1e-6)


def _dot(a, b):
    return jnp.dot(a, b, preferred_element_type=jnp.float32)


def _edge_body(e_ref, gs_ref, gr_ref, w1_ref, b1_ref, w2_ref, b2_ref,
               eo_ref, en_ref):
    pre = (_dot(e_ref[...], w1_ref[...])
           + gs_ref[...].astype(jnp.float32)
           + gr_ref[...].astype(jnp.float32) + b1_ref[...])
    h = jnp.maximum(pre, 0.0)
    en = _ln(_dot(h, w2_ref[...]) + b2_ref[...])
    en_ref[...] = en
    eo_ref[...] = e_ref[...] + en


def _edge_step(e, gs, gr, w1, b1, w2, b2):
    blk = lambda r, c: pl.BlockSpec((r, c), lambda i: (i, 0))
    cst = lambda r, c: pl.BlockSpec((r, c), lambda i: (0, 0))
    return pl.pallas_call(
        _edge_body,
        grid=(_EPAD // _BE,),
        in_specs=[blk(_BE, _L), blk(_BE, _L), blk(_BE, _L),
                  cst(_L, _L), cst(1, _L), cst(_L, _L), cst(1, _L)],
        out_specs=[blk(_BE, _L), blk(_BE, _L)],
        out_shape=[jax.ShapeDtypeStruct((_EPAD, _L), jnp.float32)] * 2,
    )(e, gs, gr, w1, b1, w2, b2)


def _enc_edge_body(sp_ref, rp_ref, w1_ref, wn_ref, b1_ref, w2_ref, b2_ref,
                   e_ref):
    rel = (sp_ref[...] - rp_ref[...]) * (1.0 / _RADIUS)
    nrm = jnp.sqrt(jnp.sum(rel * rel, axis=-1, keepdims=True))
    pre = _dot(rel, w1_ref[...]) + nrm * wn_ref[...] + b1_ref[...]
    h = jnp.maximum(pre, 0.0)
    e_ref[...] = _ln(_dot(h, w2_ref[...]) + b2_ref[...])


def _enc_edge(spos, rpos, w1p, wn, b1, w2, b2):
    blk = lambda r, c: pl.BlockSpec((r, c), lambda i: (i, 0))
    cst = lambda r, c: pl.BlockSpec((r, c), lambda i: (0, 0))
    return pl.pallas_call(
        _enc_edge_body,
        grid=(_EPAD // _BE,),
        in_specs=[blk(_BE, 16), blk(_BE, 16),
                  cst(16, _L), cst(1, _L), cst(1, _L), cst(_L, _L), cst(1, _L)],
        out_specs=blk(_BE, _L),
        out_shape=jax.ShapeDtypeStruct((_EPAD, _L), jnp.float32),
    )(spos, rpos, w1p, wn, b1, w2, b2)


def _enc_node_body(x_ref, w1_ref, b1_ref, w2_ref, b2_ref, ws_ref, wr_ref,
                   v_ref, ps_ref, pr_ref):
    h = jnp.maximum(_dot(x_ref[...], w1_ref[...]) + b1_ref[...], 0.0)
    v = _ln(_dot(h, w2_ref[...]) + b2_ref[...])
    v_ref[...] = v
    ps_ref[...] = _dot(v, ws_ref[...]).astype(jnp.bfloat16)
    pr_ref[...] = _dot(v, wr_ref[...]).astype(jnp.bfloat16)


def _enc_node(x, w1, b1, w2, b2, ws, wr):
    blk = lambda r, c: pl.BlockSpec((r, c), lambda i: (i, 0))
    cst = lambda r, c: pl.BlockSpec((r, c), lambda i: (0, 0))
    return pl.pallas_call(
        _enc_node_body,
        grid=(_N // _BN,),
        in_specs=[blk(_BN, 32), cst(32, _L), cst(1, _L), cst(_L, _L),
                  cst(1, _L), cst(_L, _L), cst(_L, _L)],
        out_specs=[blk(_BN, _L)] * 3,
        out_shape=[jax.ShapeDtypeStruct((_N, _L), jnp.float32),
                   jax.ShapeDtypeStruct((_N, _L), jnp.bfloat16),
                   jax.ShapeDtypeStruct((_N, _L), jnp.bfloat16)],
    )(x, w1, b1, w2, b2, ws, wr)


def _node_mid_body(v_ref, p0_ref, p1_ref, uv_ref, ua_ref, c1_ref, u2_ref,
                   c2_ref, ws_ref, wr_ref, vo_ref, ps_ref, pr_ref):
    v = v_ref[...]
    agg = p0_ref[...] + p1_ref[...]
    g = jnp.maximum(_dot(v, uv_ref[...]) + _dot(agg, ua_ref[...]) + c1_ref[...],
                    0.0)
    vo = v + _ln(_dot(g, u2_ref[...]) + c2_ref[...])
    vo_ref[...] = vo
    ps_ref[...] = _dot(vo, ws_ref[...]).astype(jnp.bfloat16)
    pr_ref[...] = _dot(vo, wr_ref[...]).astype(jnp.bfloat16)


def _node_mid(v, p0, p1, uv, ua, c1, u2, c2, ws, wr):
    blk = lambda r, c: pl.BlockSpec((r, c), lambda i: (i, 0))
    cst = lambda r, c: pl.BlockSpec((r, c), lambda i: (0, 0))
    return pl.pallas_call(
        _node_mid_body,
        grid=(_N // _BN,),
        in_specs=[blk(_BN, _L), blk(_BN, _L), blk(_BN, _L),
                  cst(_L, _L), cst(_L, _L), cst(1, _L), cst(_L, _L),
                  cst(1, _L), cst(_L, _L), cst(_L, _L)],
        out_specs=[blk(_BN, _L)] * 3,
        out_shape=[jax.ShapeDtypeStruct((_N, _L), jnp.float32),
                   jax.ShapeDtypeStruct((_N, _L), jnp.bfloat16),
                   jax.ShapeDtypeStruct((_N, _L), jnp.bfloat16)],
    )(v, p0, p1, uv, ua, c1, u2, c2, ws, wr)


def _node_fin_body(v_ref, p0_ref, p1_ref, uv_ref, ua_ref, c1_ref, u2_ref,
                   c2_ref, d1_ref, e1_ref, d2_ref, e2_ref, base_ref, out_ref):
    v = v_ref[...]
    agg = p0_ref[...] + p1_ref[...]
    g = jnp.maximum(_dot(v, uv_ref[...]) + _dot(agg, ua_ref[...]) + c1_ref[...],
                    0.0)
    vo = v + _ln(_dot(g, u2_ref[...]) + c2_ref[...])
    hd = jnp.maximum(_dot(vo, d1_ref[...]) + e1_ref[...], 0.0)
    acc = _dot(hd, d2_ref[...]) + e2_ref[...]
    out_ref[...] = base_ref[...] + acc


def _node_fin(v, p0, p1, uv, ua, c1, u2, c2, d1, e1, d2, e2, base):
    blk = lambda r, c: pl.BlockSpec((r, c), lambda i: (i, 0))
    cst = lambda r, c: pl.BlockSpec((r, c), lambda i: (0, 0))
    return pl.pallas_call(
        _node_fin_body,
        grid=(_N // _BN,),
        in_specs=[blk(_BN, _L), blk(_BN, _L), blk(_BN, _L),
                  cst(_L, _L), cst(_L, _L), cst(1, _L), cst(_L, _L),
                  cst(1, _L), cst(_L, _L), cst(1, _L), cst(_L, _L),
                  cst(1, _L), blk(_BN, _L)],
        out_specs=blk(_BN, _L),
        out_shape=jax.ShapeDtypeStruct((_N, _L), jnp.float32),
    )(v, p0, p1, uv, ua, c1, u2, c2, d1, e1, d2, e2, base)


def kernel(position_sequence, params, particle_types, senders, receivers,
           n_particles_per_example):
    f32 = jnp.float32
    pos = position_sequence
    most = pos[:, -1]

    nvel = (pos[:, 1:] - pos[:, :-1]).reshape(_N, -1)
    dist = jnp.concatenate([most - 0.1, 0.9 - most], axis=1)
    dist = jnp.clip(dist * (1.0 / _RADIUS), -1.0, 1.0)
    onehot = (particle_types[:, None]
              == jnp.arange(_NTYPES, dtype=particle_types.dtype)[None, :])
    x = jnp.concatenate([nvel, dist, onehot.astype(f32),
                         jnp.zeros((_N, 2), f32)], axis=1)

    a1w, a1b = params['enc_node'][0]
    a2w, a2b = params['enc_node'][1]
    wn1 = jnp.concatenate(
        [a1w[:21], params['type_emb'] @ a1w[21:37], jnp.zeros((2, _L), f32)],
        axis=0)
    b1w, b1b = params['enc_edge'][0]
    b2w, b2b = params['enc_edge'][1]
    be1 = jnp.concatenate([b1w[:3], jnp.zeros((13, _L), f32)], axis=0)
    ben = b1w[3:4]
    dw1, db1 = params['dec'][0]
    dw2, db2 = params['dec'][1]
    dw2p = jnp.concatenate([dw2, jnp.zeros((_L, _L - _DIMS), f32)], axis=1)
    db2p = jnp.concatenate([db2, jnp.zeros((_L - _DIMS,), f32)])[None, :]

    steps = []
    for sp in params['proc']:
        w1, bb1 = sp['edge'][0]
        w2, bb2 = sp['edge'][1]
        u1, cc1 = sp['node'][0]
        u2, cc2 = sp['node'][1]
        steps.append(dict(
            w1e=w1[:_L], w1s=w1[_L:2 * _L], w1r=w1[2 * _L:], b1=bb1[None, :],
            w2=w2, b2=bb2[None, :], uv=u1[:_L], ua=u1[_L:], c1=cc1[None, :],
            u2=u2, c2=cc2[None, :]))

    npad = _EPAD - _E
    i32 = jnp.int32
    s_i = senders.astype(i32)
    r_i = receivers.astype(i32)
    sidx = jnp.concatenate([s_i, jnp.zeros((npad,), i32)])
    ridx_g = jnp.concatenate([r_i, jnp.zeros((npad,), i32)])
    ridx_s = jnp.concatenate([r_i, jnp.full((npad,), _N, i32)]).reshape(-1, _IDXW)

    tpos = jnp.concatenate([most, jnp.zeros((_N, 13), f32)], axis=1)

    spos, rpos = _gather2(16, tpos, tpos, sidx, ridx_g)
    e = _enc_edge(spos, rpos, be1, ben, b1b[None, :], b2w, b2b[None, :])
    v, ps, pr = _enc_node(x, wn1, a1b[None, :], a2w, a2b[None, :],
                          steps[0]['w1s'], steps[0]['w1r'])

    base = jnp.concatenate(
        [2.0 * most - pos[:, -2], jnp.zeros((_N, _L - _DIMS), f32)], axis=1)

    for t in range(_MP_STEPS):
        st = steps[t]
        gs, gr = _gather2(_L, ps, pr, sidx, ridx_g)
        e, en = _edge_step(e, gs, gr, st['w1e'], st['b1'], st['w2'], st['b2'])
        parts = _scatter(en, ridx_s)
        if t < _MP_STEPS - 1:
            nx = steps[t + 1]
            v, ps, pr = _node_mid(v, parts[0], parts[1], st['uv'], st['ua'],
                                  st['c1'], st['u2'], st['c2'],
                                  nx['w1s'], nx['w1r'])
        else:
            out = _node_fin(v, parts[0], parts[1], st['uv'], st['ua'],
                            st['c1'], st['u2'], st['c2'], dw1, db1[None, :],
                            dw2p, db2p, base)

    return out[:, :_DIMS]
